# Initial kernel scaffold; baseline (speedup 1.0000x reference)
import jax
import jax.numpy as jnp
from jax.experimental import pallas as pl


def kernel(sample, entity_embedding, relation_embedding):
    raise NotImplementedError("write your pallas kernel here")


# speedup vs baseline: 2.0956x; 2.0956x over previous
import functools
import math

import jax
import jax.numpy as jnp
from jax import lax
from jax.experimental import pallas as pl
from jax.experimental.pallas import tpu as pltpu
from jax.experimental.pallas import tpu_sc as plsc

_GAMMA = 12.0
_EPS = 2.0
_EMB_DIM = 64
_EMB_RANGE = (_GAMMA + _EPS) / _EMB_DIM
_PI = math.pi

_B = 16384
_NC = 2
_NS = 16
_NW = _NC * _NS
_N_PER_W = _B // _NW
_CHUNK = 128


def _trig_body(rel_ref, trig_ref):
    ph = rel_ref[...] * (_PI / _EMB_RANGE)
    trig_ref[:, :_EMB_DIM] = jnp.cos(ph)
    trig_ref[:, _EMB_DIM:] = jnp.sin(ph)


def _rsqrt_newton(x):
    i = lax.bitcast_convert_type(x, jnp.int32)
    i = 0x5F3759DF - lax.shift_right_arithmetic(i, 1)
    y = lax.bitcast_convert_type(i, jnp.float32)
    for _ in range(3):
        y = y * (1.5 - 0.5 * x * y * y)
    return y


def _sc_score(hid, rid, tid, ent, trig_t):
    mesh = plsc.VectorSubcoreMesh(core_axis_name="c", subcore_axis_name="s")

    @functools.partial(
        pl.kernel,
        out_type=jax.ShapeDtypeStruct((_B,), jnp.float32),
        mesh=mesh,
        compiler_params=pltpu.CompilerParams(needs_layout_passes=False),
        scratch_types=[
            pltpu.VMEM((_CHUNK,), jnp.int32),
            pltpu.VMEM((_CHUNK,), jnp.int32),
            pltpu.VMEM((_CHUNK,), jnp.int32),
            pltpu.VMEM((_CHUNK, 128), jnp.float32),
            pltpu.VMEM((_CHUNK, 128), jnp.float32),
            pltpu.VMEM((_CHUNK, 128), jnp.float32),
            pltpu.VMEM((_CHUNK,), jnp.float32),
            pltpu.SemaphoreType.DMA,
        ],
    )
    def sc_kernel(hid_hbm, rid_hbm, tid_hbm, ent_hbm, trig_hbm,
                  out_hbm, hid_v, rid_v, tid_v, head_v, tail_v, trig_v,
                  out_v, sem):
        wid = lax.axis_index("s") * _NC + lax.axis_index("c")
        base = wid * _N_PER_W

        for c in range(_N_PER_W // _CHUNK):
            off = base + c * _CHUNK
            pltpu.sync_copy(hid_hbm.at[pl.ds(off, _CHUNK)], hid_v)
            pltpu.sync_copy(rid_hbm.at[pl.ds(off, _CHUNK)], rid_v)
            pltpu.sync_copy(tid_hbm.at[pl.ds(off, _CHUNK)], tid_v)
            pltpu.async_copy(ent_hbm.at[hid_v], head_v, sem).wait()
            pltpu.async_copy(ent_hbm.at[tid_v], tail_v, sem).wait()
            pltpu.async_copy(trig_hbm.at[rid_v], trig_v, sem).wait()

            lane = lax.iota(jnp.int32, 16)

            def group_body(g, _):
                def sample_body(j, vec):
                    s = g * 16 + j
                    acc = jnp.zeros((16,), jnp.float32)
                    for k in range(4):
                        re_h = head_v[s, pl.ds(k * 16, 16)]
                        im_h = head_v[s, pl.ds(64 + k * 16, 16)]
                        re_t = tail_v[s, pl.ds(k * 16, 16)]
                        im_t = tail_v[s, pl.ds(64 + k * 16, 16)]
                        re_r = trig_v[s, pl.ds(k * 16, 16)]
                        im_r = trig_v[s, pl.ds(64 + k * 16, 16)]
                        a = re_h * re_r - im_h * im_r - re_t
                        b = re_h * im_r + im_h * re_r - im_t
                        x = a * a + b * b
                        x = jnp.maximum(x, 1e-12)
                        acc = acc + x * _rsqrt_newton(x)
                    total = _GAMMA - jnp.sum(acc)
                    return jnp.where(lane == j, total, vec)

                vec = lax.fori_loop(0, 16, sample_body,
                                    jnp.zeros((16,), jnp.float32))
                out_v[pl.ds(g * 16, 16)] = vec
                return _

            lax.fori_loop(0, _CHUNK // 16, group_body, 0)
            pltpu.sync_copy(out_v, out_hbm.at[pl.ds(off, _CHUNK)])

    return sc_kernel(hid, rid, tid, ent, trig_t)


def kernel(sample, entity_embedding, relation_embedding):
    trig_t = pl.pallas_call(
        _trig_body,
        out_shape=jax.ShapeDtypeStruct(
            (relation_embedding.shape[0], 2 * _EMB_DIM), jnp.float32),
    )(relation_embedding)
    hid = sample[:, 0]
    rid = sample[:, 1]
    tid = sample[:, 2]
    score = _sc_score(hid, rid, tid, entity_embedding, trig_t)
    return score.reshape(_B, 1)


# speedup vs baseline: 2.3758x; 1.1337x over previous
import functools
import math

import jax
import jax.numpy as jnp
from jax import lax
from jax.experimental import pallas as pl
from jax.experimental.pallas import tpu as pltpu
from jax.experimental.pallas import tpu_sc as plsc

_GAMMA = 12.0
_EPS = 2.0
_EMB_DIM = 64
_EMB_RANGE = (_GAMMA + _EPS) / _EMB_DIM
_PI = math.pi

_B = 16384
_NC = 2
_NS = 16
_NW = _NC * _NS
_N_PER_W = _B // _NW
_CHUNK = 64
_NCHUNK = _N_PER_W // _CHUNK


def _trig_body(rel_ref, trig_ref):
    ph = rel_ref[...] * (_PI / _EMB_RANGE)
    trig_ref[:, :_EMB_DIM] = jnp.cos(ph)
    trig_ref[:, _EMB_DIM:] = jnp.sin(ph)


def _rsqrt_newton(x):
    i = lax.bitcast_convert_type(x, jnp.int32)
    i = 0x5F3759DF - lax.shift_right_arithmetic(i, 1)
    y = lax.bitcast_convert_type(i, jnp.float32)
    for _ in range(2):
        y = y * (1.5 - 0.5 * x * y * y)
    return y


def _sc_score(sample, ent, trig_t):
    mesh = plsc.VectorSubcoreMesh(core_axis_name="c", subcore_axis_name="s")

    buf = lambda shape, dt=jnp.float32: pltpu.VMEM(shape, dt)

    @functools.partial(
        pl.kernel,
        out_type=jax.ShapeDtypeStruct((_B,), jnp.float32),
        mesh=mesh,
        compiler_params=pltpu.CompilerParams(needs_layout_passes=False),
        scratch_types=[
            [buf((_CHUNK, 3), jnp.int32) for _ in range(2)],
            [buf((_CHUNK,), jnp.int32) for _ in range(2)],
            [buf((_CHUNK,), jnp.int32) for _ in range(2)],
            [buf((_CHUNK,), jnp.int32) for _ in range(2)],
            [buf((_CHUNK, 128)) for _ in range(2)],
            [buf((_CHUNK, 128)) for _ in range(2)],
            [buf((_CHUNK, 128)) for _ in range(2)],
            buf((_N_PER_W,)),
            pltpu.SemaphoreType.DMA,
            pltpu.SemaphoreType.DMA,
        ],
    )
    def sc_kernel(samp_hbm, ent_hbm, trig_hbm, out_hbm, samp_v, hid_v,
                  rid_v, tid_v, head_v, tail_v, trig_v, out_v, sem0, sem1):
        wid = lax.axis_index("s") * _NC + lax.axis_index("c")
        base = wid * _N_PER_W
        lane = lax.iota(jnp.int32, 16)
        col0 = jnp.zeros((16,), jnp.int32)
        col1 = col0 + 1
        col2 = col0 + 2
        sems = (sem0, sem1)

        def issue(c, b):
            off = base + c * _CHUNK
            pltpu.sync_copy(samp_hbm.at[pl.ds(off, _CHUNK)], samp_v[b])
            for j in range(_CHUNK // 16):
                rows = j * 16 + lane
                sl = pl.ds(j * 16, 16)
                hid_v[b][sl] = plsc.load_gather(samp_v[b], [rows, col0])
                rid_v[b][sl] = plsc.load_gather(samp_v[b], [rows, col1])
                tid_v[b][sl] = plsc.load_gather(samp_v[b], [rows, col2])
            return (
                pltpu.async_copy(ent_hbm.at[hid_v[b]], head_v[b], sems[b]),
                pltpu.async_copy(ent_hbm.at[tid_v[b]], tail_v[b], sems[b]),
                pltpu.async_copy(trig_hbm.at[rid_v[b]], trig_v[b], sems[b]),
            )

        def compute(c, b):
            head, tail, trig = head_v[b], tail_v[b], trig_v[b]

            def group_body(g, _):
                def sample_body(j, vec):
                    s = g * 16 + j
                    acc = jnp.zeros((16,), jnp.float32)
                    for k in range(4):
                        re_h = head[s, pl.ds(k * 16, 16)]
                        im_h = head[s, pl.ds(64 + k * 16, 16)]
                        re_t = tail[s, pl.ds(k * 16, 16)]
                        im_t = tail[s, pl.ds(64 + k * 16, 16)]
                        re_r = trig[s, pl.ds(k * 16, 16)]
                        im_r = trig[s, pl.ds(64 + k * 16, 16)]
                        a = re_h * re_r - im_h * im_r - re_t
                        bb = re_h * im_r + im_h * re_r - im_t
                        x = a * a + bb * bb
                        x = jnp.maximum(x, 1e-12)
                        acc = acc + x * _rsqrt_newton(x)
                    total = _GAMMA - jnp.sum(acc)
                    return jnp.where(lane == j, total, vec)

                vec = lax.fori_loop(0, 16, sample_body,
                                    jnp.zeros((16,), jnp.float32),
                                    unroll=4)
                out_v[pl.ds(c * _CHUNK + g * 16, 16)] = vec
                return _

            lax.fori_loop(0, _CHUNK // 16, group_body, 0)

        handles = issue(0, 0)
        for c in range(_NCHUNK):
            nxt = None
            if c + 1 < _NCHUNK:
                nxt = issue(c + 1, (c + 1) % 2)
            for h in handles:
                h.wait()
            compute(c, c % 2)
            handles = nxt
        pltpu.sync_copy(out_v, out_hbm.at[pl.ds(base, _N_PER_W)])

    return sc_kernel(sample, ent, trig_t)


def kernel(sample, entity_embedding, relation_embedding):
    trig_t = pl.pallas_call(
        _trig_body,
        out_shape=jax.ShapeDtypeStruct(
            (relation_embedding.shape[0], 2 * _EMB_DIM), jnp.float32),
    )(relation_embedding)
    score = _sc_score(sample, entity_embedding, trig_t)
    return score.reshape(_B, 1)
